# Initial kernel scaffold; baseline (speedup 1.0000x reference)
import jax
import jax.numpy as jnp
from jax.experimental import pallas as pl


def kernel(z_e, W):
    raise NotImplementedError("write your pallas kernel here")


# speedup vs baseline: 1.8923x; 1.8923x over previous
import jax
import jax.numpy as jnp
from jax.experimental import pallas as pl

_NUM_EMBED = 1024
_EMBED_DIM = 64
_COMMIT = 0.25


def _vq_kernel(z_ref, w_ref, zq_ref, idx_ref, sse_ref):
    b = pl.program_id(0)
    z = z_ref[0]
    w = w_ref[...]
    wsq = jnp.sum(w * w, axis=1, keepdims=True)
    zsq = jnp.sum(z * z, axis=0, keepdims=True)
    m = jax.lax.dot_general(w, z, (((1,), (0,)), ((), ())),
                            preferred_element_type=jnp.float32)
    scores = (zsq + wsq) - 2.0 * m
    minv = jnp.min(scores, axis=0, keepdims=True)
    cio = jax.lax.broadcasted_iota(jnp.int32, scores.shape, 0)
    idx = jnp.min(jnp.where(scores == minv, cio, jnp.int32(2**30)), axis=0)
    idx_ref[0, 0, :] = idx
    onehot = (cio == idx[None, :]).astype(jnp.float32)
    zq = jax.lax.dot_general(w, onehot, (((0,), (0,)), ((), ())),
                             preferred_element_type=jnp.float32)
    zq_ref[0] = z + (zq - z)
    sse = jnp.sum((zq - z) ** 2)
    tile = jnp.full((8, 128), sse, jnp.float32)

    @pl.when(b == 0)
    def _init():
        sse_ref[...] = tile

    @pl.when(b > 0)
    def _acc():
        sse_ref[...] = sse_ref[...] + tile


def kernel(z_e, W):
    B, D, H, Wd = z_e.shape
    T = H * Wd
    z3 = z_e.reshape(B, D, T)
    zq3, idx3, sse = pl.pallas_call(
        _vq_kernel,
        grid=(B,),
        in_specs=[
            pl.BlockSpec((1, D, T), lambda b: (b, 0, 0)),
            pl.BlockSpec((_NUM_EMBED, D), lambda b: (0, 0)),
        ],
        out_specs=[
            pl.BlockSpec((1, D, T), lambda b: (b, 0, 0)),
            pl.BlockSpec((1, 1, T), lambda b: (b, 0, 0)),
            pl.BlockSpec((8, 128), lambda b: (0, 0)),
        ],
        out_shape=[
            jax.ShapeDtypeStruct((B, D, T), jnp.float32),
            jax.ShapeDtypeStruct((B, 1, T), jnp.int32),
            jax.ShapeDtypeStruct((8, 128), jnp.float32),
        ],
    )(z3, W)
    z_q_st = zq3.reshape(B, D, H, Wd)
    indices = idx3.reshape(B, H, Wd)
    vq_loss = sse[0, 0] / jnp.float32(B * D * T)
    commitment_loss = jnp.float32(_COMMIT) * vq_loss
    return (z_q_st, indices, vq_loss, commitment_loss)


# speedup vs baseline: 2.2071x; 1.1664x over previous
import jax
import jax.numpy as jnp
from jax.experimental import pallas as pl
from jax.experimental.pallas import tpu as pltpu

_NUM_EMBED = 1024
_EMBED_DIM = 64
_COMMIT = 0.25


def _vq_kernel(z_ref, w_ref, zq_ref, idx_ref, sse_ref):
    z = z_ref[0]
    w = w_ref[...]
    wsq = jnp.sum(w * w, axis=1, keepdims=True)
    zsq = jnp.sum(z * z, axis=0, keepdims=True)
    mm = jax.lax.dot_general(-2.0 * w, z, (((1,), (0,)), ((), ())),
                             preferred_element_type=jnp.float32)
    scores = (zsq + wsq) + mm
    idx = jnp.argmin(scores, axis=0).astype(jnp.int32)
    idx_ref[0, 0, :] = idx
    cio = jax.lax.broadcasted_iota(jnp.int32, scores.shape, 0)
    onehot = (cio == idx[None, :]).astype(jnp.float32)
    zq = jax.lax.dot_general(w, onehot, (((0,), (0,)), ((), ())),
                             preferred_element_type=jnp.float32)
    zq_ref[0] = z + (zq - z)
    sse_ref[0] = jnp.full((8, 128), jnp.sum((zq - z) ** 2), jnp.float32)


def kernel(z_e, W):
    B, D, H, Wd = z_e.shape
    T = H * Wd
    z3 = z_e.reshape(B, D, T)
    zq3, idx3, sse = pl.pallas_call(
        _vq_kernel,
        grid=(B,),
        in_specs=[
            pl.BlockSpec((1, D, T), lambda b: (b, 0, 0)),
            pl.BlockSpec((_NUM_EMBED, D), lambda b: (0, 0)),
        ],
        out_specs=[
            pl.BlockSpec((1, D, T), lambda b: (b, 0, 0)),
            pl.BlockSpec((1, 1, T), lambda b: (b, 0, 0)),
            pl.BlockSpec((1, 8, 128), lambda b: (b, 0, 0)),
        ],
        out_shape=[
            jax.ShapeDtypeStruct((B, D, T), jnp.float32),
            jax.ShapeDtypeStruct((B, 1, T), jnp.int32),
            jax.ShapeDtypeStruct((B, 8, 128), jnp.float32),
        ],
        compiler_params=pltpu.CompilerParams(
            dimension_semantics=("parallel",),
        ),
    )(z3, W)
    z_q_st = zq3.reshape(B, D, H, Wd)
    indices = idx3.reshape(B, H, Wd)
    vq_loss = jnp.sum(sse[:, 0, 0]) / jnp.float32(B * D * T)
    commitment_loss = jnp.float32(_COMMIT) * vq_loss
    return (z_q_st, indices, vq_loss, commitment_loss)
